# baseline (speedup 1.0000x reference)
import jax
import jax.numpy as jnp
from jax.experimental import pallas as pl

_B, _H, _W, _C = 32, 14, 14, 512
_HW = _H * _W
_KM_ITERS = 11
_PB = 8
_MB = 4


def _points_kmeans_body(x_ref, perm_ref, pts_ref, mask_ref):
    i = pl.program_id(0)
    x = x_ref[...]
    colmax = jnp.max(x, axis=1)
    arg_w = jnp.argmax(colmax, axis=1)
    rowmax = jnp.max(x, axis=2)
    arg_h = jnp.argmax(rowmax, axis=1)
    pts_ref[pl.ds(i * _PB, _PB), 0, :] = arg_w.astype(jnp.float32)
    pts_ref[pl.ds(i * _PB, _PB), 1, :] = arg_h.astype(jnp.float32)

    @pl.when(i == (_B // _PB) - 1)
    def _kmeans():
        px = pts_ref[:, 0, :]
        py = pts_ref[:, 1, :]
        P = perm_ref[...]
        cx = jnp.dot(P, px[:, 0:2], preferred_element_type=jnp.float32)
        cy = jnp.dot(P, py[:, 0:2], preferred_element_type=jnp.float32)
        c0x, c1x = cx[:, 0:1], cx[:, 1:2]
        c0y, c1y = cy[:, 0:1], cy[:, 1:2]
        m1 = jnp.zeros((_B, _C), jnp.float32)
        for _ in range(_KM_ITERS):
            d0 = (px - c0x) ** 2 + (py - c0y) ** 2
            d1 = (px - c1x) ** 2 + (py - c1y) ** 2
            m1 = (d1 < d0).astype(jnp.float32)
            m0 = 1.0 - m1
            s1 = jnp.sum(m1, axis=1, keepdims=True)
            cnt1 = jnp.maximum(s1, 1.0)
            cnt0 = jnp.maximum(jnp.float32(_C) - s1, 1.0)
            c0x = jnp.sum(px * m1, axis=1, keepdims=True) / cnt1
            c0y = jnp.sum(py * m1, axis=1, keepdims=True) / cnt1
            c1x = jnp.sum(px * m0, axis=1, keepdims=True) / cnt0
            c1y = jnp.sum(py * m0, axis=1, keepdims=True) / cnt0
        mask_ref[...] = m1[:, None, :]


def _mask_body(x_ref, m_ref, c0_ref, c1_ref):
    x = x_ref[...]
    rowidx = jax.lax.broadcasted_iota(jnp.int32, (_MB * _HW, 1), 0)
    m = m_ref[0, 0, :][None, :]
    for b in range(1, _MB):
        m = jnp.where(rowidx < b * _HW, m, m_ref[b, 0, :][None, :])
    keep1 = m > 0.0
    c1_ref[...] = jnp.where(keep1, x, 0.0)
    c0_ref[...] = jnp.where(keep1, 0.0, x)


def kernel(feature_batch):
    perm = jax.random.permutation(jax.random.key(1), _B)
    P = jax.nn.one_hot(perm, _B, dtype=jnp.float32)

    _, mask = pl.pallas_call(
        _points_kmeans_body,
        grid=(_B // _PB,),
        in_specs=[pl.BlockSpec((_PB, _H, _W, _C), lambda i: (i, 0, 0, 0)),
                  pl.BlockSpec((_B, _B), lambda i: (0, 0))],
        out_specs=[pl.BlockSpec((_B, 2, _C), lambda i: (0, 0, 0)),
                   pl.BlockSpec((_B, 1, _C), lambda i: (0, 0, 0))],
        out_shape=[jax.ShapeDtypeStruct((_B, 2, _C), jnp.float32),
                   jax.ShapeDtypeStruct((_B, 1, _C), jnp.float32)],
    )(feature_batch, P)

    xf = feature_batch.reshape(_B * _HW, _C)
    c0, c1 = pl.pallas_call(
        _mask_body,
        grid=(_B // _MB,),
        in_specs=[pl.BlockSpec((_MB * _HW, _C), lambda i: (i, 0)),
                  pl.BlockSpec((_MB, 1, _C), lambda i: (i, 0, 0))],
        out_specs=[pl.BlockSpec((_MB * _HW, _C), lambda i: (i, 0)),
                   pl.BlockSpec((_MB * _HW, _C), lambda i: (i, 0))],
        out_shape=[jax.ShapeDtypeStruct((_B * _HW, _C), jnp.float32),
                   jax.ShapeDtypeStruct((_B * _HW, _C), jnp.float32)],
    )(xf, mask)
    return (c0.reshape(_B, _H, _W, _C), c1.reshape(_B, _H, _W, _C))


# speedup vs baseline: 1.4139x; 1.4139x over previous
import jax
import jax.numpy as jnp
from jax.experimental import pallas as pl

_B, _H, _W, _C = 32, 14, 14, 512
_HW = _H * _W
_KM_ITERS = 11
_PB = 8
_MB = 4


def _points_kmeans_body(x_ref, perm_ref, pts_ref, mask_ref):
    i = pl.program_id(0)
    x = x_ref[...]
    colmax = jnp.max(x, axis=1)
    arg_w = jnp.argmax(colmax, axis=1)
    rowmax = jnp.max(x, axis=2)
    arg_h = jnp.argmax(rowmax, axis=1)
    pts_ref[pl.ds(i * _PB, _PB), 0, :] = arg_w.astype(jnp.float32)
    pts_ref[pl.ds(i * _PB, _PB), 1, :] = arg_h.astype(jnp.float32)

    @pl.when(i == (_B // _PB) - 1)
    def _kmeans():
        px = pts_ref[:, 0, :]
        py = pts_ref[:, 1, :]
        P = perm_ref[...]
        cx = jnp.dot(P, px[:, 0:2], preferred_element_type=jnp.float32)
        cy = jnp.dot(P, py[:, 0:2], preferred_element_type=jnp.float32)
        c0x, c1x = cx[:, 0:1], cx[:, 1:2]
        c0y, c1y = cy[:, 0:1], cy[:, 1:2]
        m1 = jnp.zeros((_B, _C), jnp.float32)
        for _ in range(_KM_ITERS):
            d0 = (px - c0x) ** 2 + (py - c0y) ** 2
            d1 = (px - c1x) ** 2 + (py - c1y) ** 2
            m1 = (d1 < d0).astype(jnp.float32)
            m0 = 1.0 - m1
            s1 = jnp.sum(m1, axis=1, keepdims=True)
            cnt1 = jnp.maximum(s1, 1.0)
            cnt0 = jnp.maximum(jnp.float32(_C) - s1, 1.0)
            c0x = jnp.sum(px * m1, axis=1, keepdims=True) / cnt1
            c0y = jnp.sum(py * m1, axis=1, keepdims=True) / cnt1
            c1x = jnp.sum(px * m0, axis=1, keepdims=True) / cnt0
            c1y = jnp.sum(py * m0, axis=1, keepdims=True) / cnt0
        mask_ref[...] = m1[:, None, :]


def _mask_body(x_ref, m_ref, c0_ref, c1_ref):
    x = x_ref[...]
    keep1 = m_ref[...][:, None, :, :] > 0.0
    c1_ref[...] = jnp.where(keep1, x, 0.0)
    c0_ref[...] = jnp.where(keep1, 0.0, x)


def kernel(feature_batch):
    perm = jax.random.permutation(jax.random.key(1), _B)
    P = jax.nn.one_hot(perm, _B, dtype=jnp.float32)

    _, mask = pl.pallas_call(
        _points_kmeans_body,
        grid=(_B // _PB,),
        in_specs=[pl.BlockSpec((_PB, _H, _W, _C), lambda i: (i, 0, 0, 0)),
                  pl.BlockSpec((_B, _B), lambda i: (0, 0))],
        out_specs=[pl.BlockSpec((_B, 2, _C), lambda i: (0, 0, 0)),
                   pl.BlockSpec((_B, 1, _C), lambda i: (0, 0, 0))],
        out_shape=[jax.ShapeDtypeStruct((_B, 2, _C), jnp.float32),
                   jax.ShapeDtypeStruct((_B, 1, _C), jnp.float32)],
    )(feature_batch, P)

    c0, c1 = pl.pallas_call(
        _mask_body,
        grid=(_B // _MB,),
        in_specs=[pl.BlockSpec((_MB, _H, _W, _C), lambda i: (i, 0, 0, 0)),
                  pl.BlockSpec((_MB, 1, _C), lambda i: (i, 0, 0))],
        out_specs=[pl.BlockSpec((_MB, _H, _W, _C), lambda i: (i, 0, 0, 0)),
                   pl.BlockSpec((_MB, _H, _W, _C), lambda i: (i, 0, 0, 0))],
        out_shape=[jax.ShapeDtypeStruct((_B, _H, _W, _C), jnp.float32),
                   jax.ShapeDtypeStruct((_B, _H, _W, _C), jnp.float32)],
    )(feature_batch, mask)
    return (c0, c1)


# speedup vs baseline: 3.4935x; 2.4708x over previous
import jax
import jax.numpy as jnp
from jax.experimental import pallas as pl
from jax.experimental.pallas import tpu as pltpu

_B, _H, _W, _C = 32, 14, 14, 512
_KM_ITERS = 11


def _points_kmeans_body(x_ref, perm_ref, mask_ref, cm_ref, bv_ref, bh_ref):
    i = pl.program_id(0)
    x = x_ref[0]
    rowmax = jnp.max(x, axis=0)

    @pl.when(i == 0)
    def _init():
        cm_ref[...] = x
        bv_ref[...] = rowmax
        bh_ref[...] = jnp.zeros((_B, _C), jnp.float32)

    @pl.when(i > 0)
    def _update():
        cm_ref[...] = jnp.maximum(cm_ref[...], x)
        upd = rowmax > bv_ref[...]
        bv_ref[...] = jnp.where(upd, rowmax, bv_ref[...])
        bh_ref[...] = jnp.where(upd, i.astype(jnp.float32), bh_ref[...])

    @pl.when(i == _H - 1)
    def _finish():
        cm = cm_ref[...]
        best = cm[0]
        aw = jnp.zeros((_B, _C), jnp.float32)
        for w in range(1, _W):
            upd = cm[w] > best
            best = jnp.where(upd, cm[w], best)
            aw = jnp.where(upd, jnp.float32(w), aw)
        px = aw
        py = bh_ref[...]

        P = perm_ref[...]
        cx = jnp.dot(P, px[:, 0:2], preferred_element_type=jnp.float32)
        cy = jnp.dot(P, py[:, 0:2], preferred_element_type=jnp.float32)
        c0x, c1x = cx[:, 0:1], cx[:, 1:2]
        c0y, c1y = cy[:, 0:1], cy[:, 1:2]
        m1 = jnp.zeros((_B, _C), jnp.float32)
        for _ in range(_KM_ITERS):
            d0 = (px - c0x) ** 2 + (py - c0y) ** 2
            d1 = (px - c1x) ** 2 + (py - c1y) ** 2
            m1 = (d1 < d0).astype(jnp.float32)
            m0 = 1.0 - m1
            s1 = jnp.sum(m1, axis=1, keepdims=True)
            cnt1 = jnp.maximum(s1, 1.0)
            cnt0 = jnp.maximum(jnp.float32(_C) - s1, 1.0)
            c0x = jnp.sum(px * m1, axis=1, keepdims=True) / cnt1
            c0y = jnp.sum(py * m1, axis=1, keepdims=True) / cnt1
            c1x = jnp.sum(px * m0, axis=1, keepdims=True) / cnt0
            c1y = jnp.sum(py * m0, axis=1, keepdims=True) / cnt0
        mask_ref[...] = m1


def _mask_body(x_ref, m_ref, c0_ref, c1_ref):
    x = x_ref[...]
    keep1 = (m_ref[...] > 0.0)[None, None, :, :]
    c1_ref[...] = jnp.where(keep1, x, 0.0)
    c0_ref[...] = jnp.where(keep1, 0.0, x)


def kernel(feature_batch):
    xt = jnp.transpose(feature_batch, (1, 2, 0, 3))
    perm = jax.random.permutation(jax.random.key(1), _B)
    P = jax.nn.one_hot(perm, _B, dtype=jnp.float32)

    mask = pl.pallas_call(
        _points_kmeans_body,
        grid=(_H,),
        in_specs=[pl.BlockSpec((1, _W, _B, _C), lambda i: (i, 0, 0, 0)),
                  pl.BlockSpec((_B, _B), lambda i: (0, 0))],
        out_specs=pl.BlockSpec((_B, _C), lambda i: (0, 0)),
        out_shape=jax.ShapeDtypeStruct((_B, _C), jnp.float32),
        scratch_shapes=[pltpu.VMEM((_W, _B, _C), jnp.float32),
                        pltpu.VMEM((_B, _C), jnp.float32),
                        pltpu.VMEM((_B, _C), jnp.float32)],
    )(xt, P)

    c0t, c1t = pl.pallas_call(
        _mask_body,
        grid=(_H,),
        in_specs=[pl.BlockSpec((1, _W, _B, _C), lambda i: (i, 0, 0, 0)),
                  pl.BlockSpec((_B, _C), lambda i: (0, 0))],
        out_specs=[pl.BlockSpec((1, _W, _B, _C), lambda i: (i, 0, 0, 0)),
                   pl.BlockSpec((1, _W, _B, _C), lambda i: (i, 0, 0, 0))],
        out_shape=[jax.ShapeDtypeStruct((_H, _W, _B, _C), jnp.float32),
                   jax.ShapeDtypeStruct((_H, _W, _B, _C), jnp.float32)],
    )(xt, mask)
    return (jnp.transpose(c0t, (2, 0, 1, 3)), jnp.transpose(c1t, (2, 0, 1, 3)))


# speedup vs baseline: 4.2133x; 1.2060x over previous
import jax
import jax.numpy as jnp
from jax.experimental import pallas as pl
from jax.experimental.pallas import tpu as pltpu

_B, _H, _W, _C = 32, 14, 14, 512
_KM_ITERS = 11


def _points_kmeans_body(x_ref, perm_ref, mask_ref, cm_ref, bv_ref, bh_ref):
    i = pl.program_id(0)
    x = x_ref[0]
    rowmax = jnp.max(x, axis=0)

    @pl.when(i == 0)
    def _init():
        cm_ref[...] = x
        bv_ref[...] = rowmax
        bh_ref[...] = jnp.zeros((_B, _C), jnp.float32)

    @pl.when(i > 0)
    def _update():
        cm_ref[...] = jnp.maximum(cm_ref[...], x)
        upd = rowmax > bv_ref[...]
        bv_ref[...] = jnp.where(upd, rowmax, bv_ref[...])
        bh_ref[...] = jnp.where(upd, i.astype(jnp.float32), bh_ref[...])

    @pl.when(i == _H - 1)
    def _finish():
        cm = cm_ref[...]
        best = cm[0]
        aw = jnp.zeros((_B, _C), jnp.float32)
        for w in range(1, _W):
            upd = cm[w] > best
            best = jnp.where(upd, cm[w], best)
            aw = jnp.where(upd, jnp.float32(w), aw)
        px = aw
        py = bh_ref[...]

        P = perm_ref[...]
        cx = jnp.dot(P, px[:, 0:2], preferred_element_type=jnp.float32)
        cy = jnp.dot(P, py[:, 0:2], preferred_element_type=jnp.float32)
        c0x, c1x = cx[:, 0:1], cx[:, 1:2]
        c0y, c1y = cy[:, 0:1], cy[:, 1:2]
        m1 = jnp.zeros((_B, _C), jnp.float32)
        for _ in range(_KM_ITERS):
            d0 = (px - c0x) ** 2 + (py - c0y) ** 2
            d1 = (px - c1x) ** 2 + (py - c1y) ** 2
            m1 = (d1 < d0).astype(jnp.float32)
            m0 = 1.0 - m1
            s1 = jnp.sum(m1, axis=1, keepdims=True)
            cnt1 = jnp.maximum(s1, 1.0)
            cnt0 = jnp.maximum(jnp.float32(_C) - s1, 1.0)
            c0x = jnp.sum(px * m1, axis=1, keepdims=True) / cnt1
            c0y = jnp.sum(py * m1, axis=1, keepdims=True) / cnt1
            c1x = jnp.sum(px * m0, axis=1, keepdims=True) / cnt0
            c1y = jnp.sum(py * m0, axis=1, keepdims=True) / cnt0
        mask_ref[...] = m1


def _mask_body(x_ref, m_ref, c0_ref, c1_ref):
    x = x_ref[...]
    keep1 = (m_ref[...] > 0.0)[None, None, :, :]
    c1_ref[...] = jnp.where(keep1, x, 0.0)
    c0_ref[...] = jnp.where(keep1, 0.0, x)


def kernel(feature_batch):
    xt = jnp.transpose(feature_batch, (1, 2, 0, 3))
    with jax.ensure_compile_time_eval():
        perm = jax.random.permutation(jax.random.key(1), _B)
        P = jax.nn.one_hot(perm, _B, dtype=jnp.float32)

    mask = pl.pallas_call(
        _points_kmeans_body,
        grid=(_H,),
        in_specs=[pl.BlockSpec((1, _W, _B, _C), lambda i: (i, 0, 0, 0)),
                  pl.BlockSpec((_B, _B), lambda i: (0, 0))],
        out_specs=pl.BlockSpec((_B, _C), lambda i: (0, 0)),
        out_shape=jax.ShapeDtypeStruct((_B, _C), jnp.float32),
        scratch_shapes=[pltpu.VMEM((_W, _B, _C), jnp.float32),
                        pltpu.VMEM((_B, _C), jnp.float32),
                        pltpu.VMEM((_B, _C), jnp.float32)],
    )(xt, P)

    c0t, c1t = pl.pallas_call(
        _mask_body,
        grid=(_H,),
        in_specs=[pl.BlockSpec((1, _W, _B, _C), lambda i: (i, 0, 0, 0)),
                  pl.BlockSpec((_B, _C), lambda i: (0, 0))],
        out_specs=[pl.BlockSpec((1, _W, _B, _C), lambda i: (i, 0, 0, 0)),
                   pl.BlockSpec((1, _W, _B, _C), lambda i: (i, 0, 0, 0))],
        out_shape=[jax.ShapeDtypeStruct((_H, _W, _B, _C), jnp.float32),
                   jax.ShapeDtypeStruct((_H, _W, _B, _C), jnp.float32)],
    )(xt, mask)
    return (jnp.transpose(c0t, (2, 0, 1, 3)), jnp.transpose(c1t, (2, 0, 1, 3)))


# speedup vs baseline: 4.3043x; 1.0216x over previous
import jax
import jax.numpy as jnp
from jax.experimental import pallas as pl
from jax.experimental.pallas import tpu as pltpu

_B, _H, _W, _C = 32, 14, 14, 512
_KM_ITERS = 11


def _fused_body(x_ref, perm_ref, c0_ref, c1_ref,
                cm_ref, bv_ref, bh_ref, mask_ref):
    i = pl.program_id(0)
    x = x_ref[0]

    @pl.when(i == 0)
    def _init():
        cm_ref[...] = x
        bv_ref[...] = jnp.max(x, axis=0)
        bh_ref[...] = jnp.zeros((_B, _C), jnp.float32)

    @pl.when((i > 0) & (i < _H))
    def _update():
        rowmax = jnp.max(x, axis=0)
        cm_ref[...] = jnp.maximum(cm_ref[...], x)
        upd = rowmax > bv_ref[...]
        bv_ref[...] = jnp.where(upd, rowmax, bv_ref[...])
        bh_ref[...] = jnp.where(upd, i.astype(jnp.float32), bh_ref[...])

    @pl.when(i == _H - 1)
    def _kmeans():
        cm = cm_ref[...]
        best = cm[0]
        aw = jnp.zeros((_B, _C), jnp.float32)
        for w in range(1, _W):
            upd = cm[w] > best
            best = jnp.where(upd, cm[w], best)
            aw = jnp.where(upd, jnp.float32(w), aw)
        px = aw
        py = bh_ref[...]

        P = perm_ref[...]
        cx = jnp.dot(P, px[:, 0:2], preferred_element_type=jnp.float32)
        cy = jnp.dot(P, py[:, 0:2], preferred_element_type=jnp.float32)
        c0x, c1x = cx[:, 0:1], cx[:, 1:2]
        c0y, c1y = cy[:, 0:1], cy[:, 1:2]
        m1 = jnp.zeros((_B, _C), jnp.float32)
        for _ in range(_KM_ITERS):
            d0 = (px - c0x) ** 2 + (py - c0y) ** 2
            d1 = (px - c1x) ** 2 + (py - c1y) ** 2
            m1 = (d1 < d0).astype(jnp.float32)
            m0 = 1.0 - m1
            s1 = jnp.sum(m1, axis=1, keepdims=True)
            cnt1 = jnp.maximum(s1, 1.0)
            cnt0 = jnp.maximum(jnp.float32(_C) - s1, 1.0)
            c0x = jnp.sum(px * m1, axis=1, keepdims=True) / cnt1
            c0y = jnp.sum(py * m1, axis=1, keepdims=True) / cnt1
            c1x = jnp.sum(px * m0, axis=1, keepdims=True) / cnt0
            c1y = jnp.sum(py * m0, axis=1, keepdims=True) / cnt0
        mask_ref[...] = m1

    @pl.when(i >= _H)
    def _split():
        keep1 = (mask_ref[...] > 0.0)[None, :, :]
        c1_ref[0] = jnp.where(keep1, x, 0.0)
        c0_ref[0] = jnp.where(keep1, 0.0, x)


def kernel(feature_batch):
    xt = jnp.transpose(feature_batch, (1, 2, 0, 3))
    with jax.ensure_compile_time_eval():
        perm = jax.random.permutation(jax.random.key(1), _B)
        P = jax.nn.one_hot(perm, _B, dtype=jnp.float32)

    out_spec = pl.BlockSpec((1, _W, _B, _C),
                            lambda i: (jnp.maximum(i - _H, 0), 0, 0, 0))
    c0t, c1t = pl.pallas_call(
        _fused_body,
        grid=(2 * _H,),
        in_specs=[pl.BlockSpec((1, _W, _B, _C), lambda i: (i % _H, 0, 0, 0)),
                  pl.BlockSpec((_B, _B), lambda i: (0, 0))],
        out_specs=[out_spec, out_spec],
        out_shape=[jax.ShapeDtypeStruct((_H, _W, _B, _C), jnp.float32),
                   jax.ShapeDtypeStruct((_H, _W, _B, _C), jnp.float32)],
        scratch_shapes=[pltpu.VMEM((_W, _B, _C), jnp.float32),
                        pltpu.VMEM((_B, _C), jnp.float32),
                        pltpu.VMEM((_B, _C), jnp.float32),
                        pltpu.VMEM((_B, _C), jnp.float32)],
    )(xt, P)
    return (jnp.transpose(c0t, (2, 0, 1, 3)), jnp.transpose(c1t, (2, 0, 1, 3)))


# speedup vs baseline: 5.5278x; 1.2842x over previous
import jax
import jax.numpy as jnp
from jax.experimental import pallas as pl
from jax.experimental.pallas import tpu as pltpu

_B, _H, _W, _C = 32, 14, 14, 512
_KM_ITERS = 11


def _fused_body(x_ref, perm_ref, c0_ref, c1_ref,
                cm_ref, bv_ref, bh_ref, mask_ref, xs_ref):
    i = pl.program_id(0)

    @pl.when(i == 0)
    def _init():
        x = x_ref[0]
        xs_ref[0] = x
        cm_ref[...] = x
        bv_ref[...] = jnp.max(x, axis=0)
        bh_ref[...] = jnp.zeros((_B, _C), jnp.float32)

    @pl.when((i > 0) & (i < _H))
    def _update():
        x = x_ref[0]
        xs_ref[i] = x
        rowmax = jnp.max(x, axis=0)
        cm_ref[...] = jnp.maximum(cm_ref[...], x)
        upd = rowmax > bv_ref[...]
        bv_ref[...] = jnp.where(upd, rowmax, bv_ref[...])
        bh_ref[...] = jnp.where(upd, i.astype(jnp.float32), bh_ref[...])

    @pl.when(i == _H - 1)
    def _kmeans():
        cm = cm_ref[...]
        best = cm[0]
        aw = jnp.zeros((_B, _C), jnp.float32)
        for w in range(1, _W):
            upd = cm[w] > best
            best = jnp.where(upd, cm[w], best)
            aw = jnp.where(upd, jnp.float32(w), aw)
        px = aw
        py = bh_ref[...]

        P = perm_ref[...]
        cx = jnp.dot(P, px[:, 0:2], preferred_element_type=jnp.float32)
        cy = jnp.dot(P, py[:, 0:2], preferred_element_type=jnp.float32)
        c0x, c1x = cx[:, 0:1], cx[:, 1:2]
        c0y, c1y = cy[:, 0:1], cy[:, 1:2]
        m1 = jnp.zeros((_B, _C), jnp.float32)
        for _ in range(_KM_ITERS):
            d0 = (px - c0x) ** 2 + (py - c0y) ** 2
            d1 = (px - c1x) ** 2 + (py - c1y) ** 2
            m1 = (d1 < d0).astype(jnp.float32)
            m0 = 1.0 - m1
            s1 = jnp.sum(m1, axis=1, keepdims=True)
            cnt1 = jnp.maximum(s1, 1.0)
            cnt0 = jnp.maximum(jnp.float32(_C) - s1, 1.0)
            c0x = jnp.sum(px * m1, axis=1, keepdims=True) / cnt1
            c0y = jnp.sum(py * m1, axis=1, keepdims=True) / cnt1
            c1x = jnp.sum(px * m0, axis=1, keepdims=True) / cnt0
            c1y = jnp.sum(py * m0, axis=1, keepdims=True) / cnt0
        mask_ref[...] = m1

    @pl.when(i >= _H)
    def _split():
        x = xs_ref[i - _H]
        keep1 = (mask_ref[...] > 0.0)[None, :, :]
        c1_ref[0] = jnp.where(keep1, x, 0.0)
        c0_ref[0] = jnp.where(keep1, 0.0, x)


def kernel(feature_batch):
    xt = jnp.transpose(feature_batch, (1, 2, 0, 3))
    with jax.ensure_compile_time_eval():
        perm = jax.random.permutation(jax.random.key(1), _B)
        P = jax.nn.one_hot(perm, _B, dtype=jnp.float32)

    out_spec = pl.BlockSpec((1, _W, _B, _C),
                            lambda i: (jnp.maximum(i - _H, 0), 0, 0, 0))
    c0t, c1t = pl.pallas_call(
        _fused_body,
        grid=(2 * _H,),
        in_specs=[pl.BlockSpec((1, _W, _B, _C),
                               lambda i: (jnp.minimum(i, _H - 1), 0, 0, 0)),
                  pl.BlockSpec((_B, _B), lambda i: (0, 0))],
        out_specs=[out_spec, out_spec],
        out_shape=[jax.ShapeDtypeStruct((_H, _W, _B, _C), jnp.float32),
                   jax.ShapeDtypeStruct((_H, _W, _B, _C), jnp.float32)],
        scratch_shapes=[pltpu.VMEM((_W, _B, _C), jnp.float32),
                        pltpu.VMEM((_B, _C), jnp.float32),
                        pltpu.VMEM((_B, _C), jnp.float32),
                        pltpu.VMEM((_B, _C), jnp.float32),
                        pltpu.VMEM((_H, _W, _B, _C), jnp.float32)],
    )(xt, P)
    return (jnp.transpose(c0t, (2, 0, 1, 3)), jnp.transpose(c1t, (2, 0, 1, 3)))


# speedup vs baseline: 5.9647x; 1.0790x over previous
import jax
import jax.numpy as jnp
from jax.experimental import pallas as pl
from jax.experimental.pallas import tpu as pltpu

_B, _H, _W, _C = 32, 14, 14, 512
_KM_ITERS = 11
_SPLIT_H = 2


def _fused_body(x_ref, perm_ref, c0_ref, c1_ref,
                cm_ref, bv_ref, bh_ref, mask_ref, xs_ref):
    i = pl.program_id(0)

    @pl.when(i == 0)
    def _init():
        x = x_ref[0]
        xs_ref[0] = x
        cm_ref[...] = x
        bv_ref[...] = jnp.max(x, axis=0)
        bh_ref[...] = jnp.zeros((_B, _C), jnp.float32)

    @pl.when((i > 0) & (i < _H))
    def _update():
        x = x_ref[0]
        xs_ref[i] = x
        rowmax = jnp.max(x, axis=0)
        cm_ref[...] = jnp.maximum(cm_ref[...], x)
        upd = rowmax > bv_ref[...]
        bv_ref[...] = jnp.where(upd, rowmax, bv_ref[...])
        bh_ref[...] = jnp.where(upd, i.astype(jnp.float32), bh_ref[...])

    @pl.when(i == _H - 1)
    def _kmeans():
        cm = cm_ref[...]
        best = cm[0]
        aw = jnp.zeros((_B, _C), jnp.float32)
        for w in range(1, _W):
            upd = cm[w] > best
            best = jnp.where(upd, cm[w], best)
            aw = jnp.where(upd, jnp.float32(w), aw)
        px = aw
        py = bh_ref[...]

        P = perm_ref[...]
        cx = jnp.dot(P, px[:, 0:2], preferred_element_type=jnp.float32)
        cy = jnp.dot(P, py[:, 0:2], preferred_element_type=jnp.float32)
        c0x, c1x = cx[:, 0:1], cx[:, 1:2]
        c0y, c1y = cy[:, 0:1], cy[:, 1:2]
        m1 = jnp.zeros((_B, _C), jnp.float32)
        for _ in range(_KM_ITERS):
            d0 = (px - c0x) ** 2 + (py - c0y) ** 2
            d1 = (px - c1x) ** 2 + (py - c1y) ** 2
            m1 = (d1 < d0).astype(jnp.float32)
            m0 = 1.0 - m1
            s1 = jnp.sum(m1, axis=1, keepdims=True)
            cnt1 = jnp.maximum(s1, 1.0)
            cnt0 = jnp.maximum(jnp.float32(_C) - s1, 1.0)
            c0x = jnp.sum(px * m1, axis=1, keepdims=True) / cnt1
            c0y = jnp.sum(py * m1, axis=1, keepdims=True) / cnt1
            c1x = jnp.sum(px * m0, axis=1, keepdims=True) / cnt0
            c1y = jnp.sum(py * m0, axis=1, keepdims=True) / cnt0
        mask_ref[...] = m1

    @pl.when(i >= _H)
    def _split():
        x = xs_ref[pl.ds((i - _H) * _SPLIT_H, _SPLIT_H)]
        keep1 = (mask_ref[...] > 0.0)[None, None, :, :]
        c1_ref[...] = jnp.where(keep1, x, 0.0)
        c0_ref[...] = jnp.where(keep1, 0.0, x)


def kernel(feature_batch):
    xt = jnp.transpose(feature_batch, (1, 2, 0, 3))
    with jax.ensure_compile_time_eval():
        perm = jax.random.permutation(jax.random.key(1), _B)
        P = jax.nn.one_hot(perm, _B, dtype=jnp.float32)

    out_spec = pl.BlockSpec((_SPLIT_H, _W, _B, _C),
                            lambda i: (jnp.maximum(i - _H, 0), 0, 0, 0))
    c0t, c1t = pl.pallas_call(
        _fused_body,
        grid=(_H + _H // _SPLIT_H,),
        in_specs=[pl.BlockSpec((1, _W, _B, _C),
                               lambda i: (jnp.minimum(i, _H - 1), 0, 0, 0)),
                  pl.BlockSpec((_B, _B), lambda i: (0, 0))],
        out_specs=[out_spec, out_spec],
        out_shape=[jax.ShapeDtypeStruct((_H, _W, _B, _C), jnp.float32),
                   jax.ShapeDtypeStruct((_H, _W, _B, _C), jnp.float32)],
        scratch_shapes=[pltpu.VMEM((_W, _B, _C), jnp.float32),
                        pltpu.VMEM((_B, _C), jnp.float32),
                        pltpu.VMEM((_B, _C), jnp.float32),
                        pltpu.VMEM((_B, _C), jnp.float32),
                        pltpu.VMEM((_H, _W, _B, _C), jnp.float32)],
    )(xt, P)
    return (jnp.transpose(c0t, (2, 0, 1, 3)), jnp.transpose(c1t, (2, 0, 1, 3)))
